# baseline (device time: 33280 ns/iter reference)
import jax
import jax.numpy as jnp
from jax import lax
from jax.experimental import pallas as pl
from jax.experimental.pallas import tpu as pltpu


def kernel(O, Wo):
    B, S, Hl, D = O.shape
    K = Hl * D
    N = Wo.shape[1]
    S_out = S // 2
    S_c = S_out // 2
    NC = 2 * B

    def body(o_ref, w_ref, out_ref, o_vmem, w_vmem, out_vmem,
             send_buf, recv_buf, send_sem, recv_sem, ld_sems, st_sems):
        my_x = lax.axis_index("x")
        my_y = lax.axis_index("y")
        my_z = lax.axis_index("z")
        peer_z = 1 - my_z

        ld_w = pltpu.make_async_copy(w_ref, w_vmem, ld_sems.at[0])
        ld_w.start()
        ld_peer = pltpu.make_async_copy(
            o_ref.at[:, pl.ds(peer_z * S_out, S_out), :],
            o_vmem.at[:, pl.ds(0, S_out), :],
            ld_sems.at[1],
        )
        ld_peer.start()
        ld_own = pltpu.make_async_copy(
            o_ref.at[:, pl.ds(my_z * S_out, S_out), :],
            o_vmem.at[:, pl.ds(S_out, S_out), :],
            ld_sems.at[2],
        )
        ld_own.start()

        barrier_sem = pltpu.get_barrier_semaphore()
        pl.semaphore_signal(
            barrier_sem, inc=1,
            device_id=(my_x, my_y, peer_z),
            device_id_type=pl.DeviceIdType.MESH,
        )
        pl.semaphore_wait(barrier_sem, 1)

        ld_w.wait()
        ld_peer.wait()
        w = w_vmem[...]

        rdmas = []
        for c in range(NC):
            b, h = c // 2, c % 2
            a = o_vmem[b, pl.ds(h * S_c, S_c), :]
            r = jnp.dot(a, w, preferred_element_type=jnp.float32)
            send_buf[b, pl.ds(h * S_c, S_c), :] = r.astype(jnp.bfloat16)
            rdma = pltpu.make_async_remote_copy(
                src_ref=send_buf.at[b, pl.ds(h * S_c, S_c)],
                dst_ref=recv_buf.at[b, pl.ds(h * S_c, S_c)],
                send_sem=send_sem.at[c],
                recv_sem=recv_sem.at[c],
                device_id=(my_x, my_y, peer_z),
                device_id_type=pl.DeviceIdType.MESH,
            )
            rdma.start()
            rdmas.append(rdma)

        ld_own.wait()
        for b in range(B):
            a = o_vmem[b, pl.ds(S_out, S_out), :]
            out_vmem[b, :, :] = jnp.dot(
                a, w, preferred_element_type=jnp.float32
            )

        stores = []
        for c in range(NC):
            b, h = c // 2, c % 2
            rdmas[c].wait_recv()
            out_vmem[b, pl.ds(h * S_c, S_c), :] = (
                out_vmem[b, pl.ds(h * S_c, S_c), :]
                + recv_buf[b, pl.ds(h * S_c, S_c), :].astype(jnp.float32)
            )
            st = pltpu.make_async_copy(
                out_vmem.at[b, pl.ds(h * S_c, S_c), :],
                out_ref.at[b, pl.ds(h * S_c, S_c), :],
                st_sems.at[c],
            )
            st.start()
            stores.append(st)

        for st in stores:
            st.wait()
        for c in range(NC):
            rdmas[c].wait_send()

    O2 = O.reshape(B, S, K)
    return pl.pallas_call(
        body,
        out_shape=jax.ShapeDtypeStruct((B, S_out, N), jnp.float32),
        in_specs=[
            pl.BlockSpec(memory_space=pltpu.HBM),
            pl.BlockSpec(memory_space=pltpu.HBM),
        ],
        out_specs=pl.BlockSpec(memory_space=pltpu.HBM),
        scratch_shapes=[
            pltpu.VMEM((B, S, K), jnp.float32),
            pltpu.VMEM((K, N), jnp.float32),
            pltpu.VMEM((B, S_out, N), jnp.float32),
            pltpu.VMEM((B, S_out, N), jnp.bfloat16),
            pltpu.VMEM((B, S_out, N), jnp.bfloat16),
            pltpu.SemaphoreType.DMA((NC,)),
            pltpu.SemaphoreType.DMA((NC,)),
            pltpu.SemaphoreType.DMA((3,)),
            pltpu.SemaphoreType.DMA((NC,)),
        ],
        compiler_params=pltpu.CompilerParams(collective_id=0),
    )(O2, Wo)


# device time: 32957 ns/iter; 1.0098x vs baseline; 1.0098x over previous
import jax
import jax.numpy as jnp
from jax import lax
from jax.experimental import pallas as pl
from jax.experimental.pallas import tpu as pltpu


def kernel(O, Wo):
    B, S, Hl, D = O.shape
    K = Hl * D
    N = Wo.shape[1]
    S_out = S // 2
    CPB = 4
    S_c = S_out // CPB
    NC = CPB * B

    def body(o_ref, w_ref, out_ref, out_vmem, send_buf, recv_buf,
             send_sem, recv_sem, st_sems):
        my_x = lax.axis_index("x")
        my_y = lax.axis_index("y")
        my_z = lax.axis_index("z")
        peer_z = 1 - my_z

        barrier_sem = pltpu.get_barrier_semaphore()
        pl.semaphore_signal(
            barrier_sem, inc=1,
            device_id=(my_x, my_y, peer_z),
            device_id_type=pl.DeviceIdType.MESH,
        )
        pl.semaphore_wait(barrier_sem, 1)

        w = w_ref[...]

        rdmas = []
        for c in range(NC):
            b, h = c // CPB, c % CPB
            a = o_ref[b, pl.ds(peer_z * S_out + h * S_c, S_c), :]
            r = jnp.dot(a, w, preferred_element_type=jnp.float32)
            send_buf[b, pl.ds(h * S_c, S_c), :] = r.astype(jnp.bfloat16)
            rdma = pltpu.make_async_remote_copy(
                src_ref=send_buf.at[b, pl.ds(h * S_c, S_c)],
                dst_ref=recv_buf.at[b, pl.ds(h * S_c, S_c)],
                send_sem=send_sem.at[c],
                recv_sem=recv_sem.at[c],
                device_id=(my_x, my_y, peer_z),
                device_id_type=pl.DeviceIdType.MESH,
            )
            rdma.start()
            rdmas.append(rdma)

        for b in range(B):
            a = o_ref[b, pl.ds(my_z * S_out, S_out), :]
            out_vmem[b, :, :] = jnp.dot(
                a, w, preferred_element_type=jnp.float32
            )

        stores = []
        for c in range(NC):
            b, h = c // CPB, c % CPB
            rdmas[c].wait_recv()
            out_vmem[b, pl.ds(h * S_c, S_c), :] = (
                out_vmem[b, pl.ds(h * S_c, S_c), :]
                + recv_buf[b, pl.ds(h * S_c, S_c), :].astype(jnp.float32)
            )
            st = pltpu.make_async_copy(
                out_vmem.at[b, pl.ds(h * S_c, S_c), :],
                out_ref.at[b, pl.ds(h * S_c, S_c), :],
                st_sems.at[c],
            )
            st.start()
            stores.append(st)

        for st in stores:
            st.wait()
        for c in range(NC):
            rdmas[c].wait_send()

    O2 = O.reshape(B, S, K)
    return pl.pallas_call(
        body,
        out_shape=jax.ShapeDtypeStruct((B, S_out, N), jnp.float32),
        in_specs=[
            pl.BlockSpec(memory_space=pltpu.VMEM),
            pl.BlockSpec(memory_space=pltpu.VMEM),
        ],
        out_specs=pl.BlockSpec(memory_space=pltpu.HBM),
        scratch_shapes=[
            pltpu.VMEM((B, S_out, N), jnp.float32),
            pltpu.VMEM((B, S_out, N), jnp.bfloat16),
            pltpu.VMEM((B, S_out, N), jnp.bfloat16),
            pltpu.SemaphoreType.DMA((NC,)),
            pltpu.SemaphoreType.DMA((NC,)),
            pltpu.SemaphoreType.DMA((NC,)),
        ],
        compiler_params=pltpu.CompilerParams(collective_id=0),
    )(O2, Wo)
